# Initial kernel scaffold; baseline (speedup 1.0000x reference)
import jax
import jax.numpy as jnp
from jax.experimental import pallas as pl


def kernel(x, edge_index, adj_values, W1, b1, W2, b2, Wc, bc):
    raise NotImplementedError("write your pallas kernel here")


# speedup vs baseline: 4.2365x; 4.2365x over previous
import functools

import jax
import jax.numpy as jnp
from jax import lax
from jax.experimental import pallas as pl
from jax.experimental.pallas import tpu as pltpu
from jax.experimental.pallas import tpu_sc as plsc

N = 10000
E = 320000
D = 128

NC = 2
NS = 16
NW = NC * NS
EDGES_PER_W = E // NW
CHUNK = 80
NCHUNKS = EDGES_PER_W // CHUNK
ROWS_PER_S = N // NS
ZROWS = 125


def _spmm_body(h_hbm, row_hbm, col_hbm, val_hbm, out_hbm,
               acc, colv, rowv, valv, rows_v, zbuf, sem):
    cid = lax.axis_index("c")
    sid = lax.axis_index("s")
    w = cid * NS + sid

    def zero_body(i, _):
        for j in range(D // 16):
            zbuf[i, pl.ds(j * 16, 16)] = jnp.zeros((16,), jnp.float32)
        return 0

    lax.fori_loop(0, ZROWS, zero_body, 0)
    for k in range(ROWS_PER_S // ZROWS):
        pltpu.sync_copy(zbuf, acc.at[pl.ds(sid * ROWS_PER_S + k * ZROWS, ZROWS)])
    plsc.subcore_barrier()

    def chunk_body(i, _):
        base = w * EDGES_PER_W + i * CHUNK
        pltpu.sync_copy(col_hbm.at[pl.ds(base, CHUNK)], colv)
        pltpu.sync_copy(row_hbm.at[pl.ds(base, CHUNK)], rowv)
        pltpu.sync_copy(val_hbm.at[pl.ds(base, CHUNK)], valv)
        pltpu.async_copy(h_hbm.at[colv], rows_v, sem).wait()

        def scale_body(g, _):
            vv = valv[pl.ds(g * 16, 16)]
            for l in range(16):
                e = g * 16 + l
                v = vv[l]
                for j in range(D // 16):
                    rows_v[e, pl.ds(j * 16, 16)] = rows_v[e, pl.ds(j * 16, 16)] * v
            return 0

        lax.fori_loop(0, CHUNK // 16, scale_body, 0)
        pltpu.sync_copy(rows_v, acc.at[rowv], add=True)
        return 0

    lax.fori_loop(0, NCHUNKS, chunk_body, 0)
    plsc.subcore_barrier()

    off = pl.multiple_of(sid * 624, 8)
    pltpu.sync_copy(acc.at[pl.ds(off, 624)], out_hbm.at[cid, pl.ds(off, 624)])

    @pl.when(sid == 0)
    def _():
        pltpu.sync_copy(acc.at[pl.ds(NS * 624, N - NS * 624)],
                        out_hbm.at[cid, pl.ds(NS * 624, N - NS * 624)])


@jax.jit
def _spmm_sc(h, row, col, vals):
    mesh = plsc.VectorSubcoreMesh(core_axis_name="c", subcore_axis_name="s")
    return pl.kernel(
        _spmm_body,
        mesh=mesh,
        out_type=jax.ShapeDtypeStruct((NC, N, D), jnp.float32),
        scratch_types=[
            pltpu.VMEM_SHARED((N, D), jnp.float32),
            pltpu.VMEM((CHUNK,), jnp.int32),
            pltpu.VMEM((CHUNK,), jnp.int32),
            pltpu.VMEM((CHUNK,), jnp.float32),
            pltpu.VMEM((CHUNK, D), jnp.float32),
            pltpu.VMEM((ZROWS, D), jnp.float32),
            pltpu.SemaphoreType.DMA,
        ],
    )(h, row, col, vals)


def _dense_body(h_ref, w_ref, b_ref, o_ref, *, act, sum2):
    h = h_ref[...]
    if sum2:
        h = h[0] + h[1]
    if act:
        h = jnp.maximum(h, 0.0)
    o_ref[...] = (jnp.dot(h, w_ref[...], preferred_element_type=jnp.float32)
                  + b_ref[...])


def _dense_tc(h, w, b, act, sum2):
    n = h.shape[-2]
    return pl.pallas_call(
        functools.partial(_dense_body, act=act, sum2=sum2),
        out_shape=jax.ShapeDtypeStruct((n, w.shape[1]), jnp.float32),
    )(h, w, b.reshape(1, -1))


def kernel(x, edge_index, adj_values, W1, b1, W2, b2, Wc, bc):
    row = edge_index[0].astype(jnp.int32)
    col = edge_index[1].astype(jnp.int32)
    h = _dense_tc(x, W1, b1, act=False, sum2=False)
    p = _spmm_sc(h, row, col, adj_values)
    h2 = _dense_tc(p, W2, b2, act=True, sum2=True)
    q = _spmm_sc(h2, row, col, adj_values)
    return _dense_tc(q, Wc, bc, act=False, sum2=True)
